# Initial kernel scaffold; baseline (speedup 1.0000x reference)
import jax
import jax.numpy as jnp
from jax.experimental import pallas as pl


def kernel(x, W1, b1, Wmu, bmu, Wls, bls, edge_index):
    raise NotImplementedError("write your pallas kernel here")


# speedup vs baseline: 16.4913x; 16.4913x over previous
import functools

import jax
import jax.numpy as jnp
from jax import lax
from jax.experimental import pallas as pl
from jax.experimental.pallas import tpu as pltpu
from jax.experimental.pallas import tpu_sc as plsc

N = 10000
E = 320000
D = 128
NC = 2
NS = 16
NW = NC * NS
EPW = E // NW
K = 80
NCHUNK = EPW // K
NP = 10240
RPT = NP // NS
ZROWS = 128
BM = 1000

_mesh = plsc.VectorSubcoreMesh(core_axis_name="c", subcore_axis_name="s")


def _ramp_into(ridx_v, off):
    @pl.loop(0, ZROWS // 16)
    def _ramp(jj):
        ridx_v[pl.ds(jj * 16, 16)] = lax.iota(jnp.int32, 16) + (off + jj * 16)


@functools.partial(
    pl.kernel,
    out_type=jax.ShapeDtypeStruct((NC, NP, 16), jnp.float32),
    mesh=_mesh,
    scratch_types=[
        pltpu.VMEM((K, 16), jnp.float32),
        pltpu.VMEM((ZROWS, 16), jnp.float32),
        pltpu.VMEM((ZROWS,), jnp.int32),
        pltpu.VMEM((NCHUNK, K), jnp.int32),
        pltpu.VMEM_SHARED((NP, 16), jnp.float32),
        pltpu.SemaphoreType.DMA,
    ],
)
def _deg_kernel(dst_hbm, out_hbm, ones_v, buf_v, ridx_v, didx_v, acc_sh, sem):
    c = lax.axis_index("c")
    s = lax.axis_index("s")
    wid = c * NS + s

    pltpu.sync_copy(dst_hbm.at[wid], didx_v)

    @pl.loop(0, K)
    def _fill_ones(j):
        ones_v[j] = jnp.full((16,), 1.0, jnp.float32)

    @pl.loop(0, ZROWS)
    def _fill_zero(j):
        buf_v[j] = jnp.zeros((16,), jnp.float32)

    @pl.loop(0, RPT // ZROWS)
    def _zero_acc(j):
        _ramp_into(ridx_v, s * RPT + j * ZROWS)
        pltpu.sync_copy(buf_v, acc_sh.at[ridx_v])

    plsc.subcore_barrier()

    @pl.loop(0, NCHUNK)
    def _chunk(ci):
        pltpu.sync_copy(ones_v, acc_sh.at[didx_v.at[ci]], add=True)

    plsc.subcore_barrier()

    @pl.loop(0, RPT // ZROWS)
    def _writeout(j):
        _ramp_into(ridx_v, s * RPT + j * ZROWS)
        pltpu.async_copy(acc_sh.at[ridx_v], buf_v, sem).wait()
        pltpu.sync_copy(buf_v, out_hbm.at[c, pl.ds(s * RPT + j * ZROWS, ZROWS)])


DH = D // 2


@functools.partial(
    pl.kernel,
    out_type=jax.ShapeDtypeStruct((NC, NP, DH), jnp.float32),
    mesh=_mesh,
    scratch_types=[
        pltpu.VMEM((NCHUNK, K), jnp.int32),
        pltpu.VMEM((NCHUNK, K), jnp.int32),
        pltpu.VMEM((2, K, DH), jnp.float32),
        pltpu.VMEM((ZROWS, DH), jnp.float32),
        pltpu.VMEM((ZROWS,), jnp.int32),
        pltpu.VMEM_SHARED((NP, DH), jnp.float32),
        pltpu.SemaphoreType.DMA,
    ],
    compiler_params=pltpu.CompilerParams(use_tc_tiling_on_sc=False),
)
def _agg_kernel(h_hbm, src_hbm, dst_hbm, out_hbm,
                sidx_v, didx_v, rows_v, buf_v, ridx_v, acc_sh, sem):
    c = lax.axis_index("c")
    s = lax.axis_index("s")
    wid = c * NS + s

    pltpu.sync_copy(src_hbm.at[wid], sidx_v)
    pltpu.sync_copy(dst_hbm.at[wid], didx_v)

    @pl.loop(0, ZROWS)
    def _fill_zero(j):
        @pl.loop(0, DH // 16)
        def _fill_row(jj):
            buf_v[j, pl.ds(jj * 16, 16)] = jnp.zeros((16,), jnp.float32)

    @pl.loop(0, RPT // ZROWS)
    def _zero_acc(j):
        _ramp_into(ridx_v, s * RPT + j * ZROWS)
        pltpu.sync_copy(buf_v, acc_sh.at[ridx_v])

    plsc.subcore_barrier()

    @pl.loop(0, NCHUNK // 2)
    def _chunk(ci2):
        for b in range(2):
            ci = ci2 * 2 + b
            pltpu.async_copy(h_hbm.at[sidx_v.at[ci]], rows_v.at[b], sem).wait()
            pltpu.sync_copy(rows_v.at[b], acc_sh.at[didx_v.at[ci]], add=True)

    _ci = NCHUNK - 1
    pltpu.async_copy(h_hbm.at[sidx_v.at[_ci]], rows_v.at[0], sem).wait()
    pltpu.sync_copy(rows_v.at[0], acc_sh.at[didx_v.at[_ci]], add=True)

    plsc.subcore_barrier()

    @pl.loop(0, RPT // ZROWS)
    def _writeout(j):
        _ramp_into(ridx_v, s * RPT + j * ZROWS)
        pltpu.async_copy(acc_sh.at[ridx_v], buf_v, sem).wait()
        pltpu.sync_copy(buf_v, out_hbm.at[c, pl.ds(s * RPT + j * ZROWS, ZROWS)])


def _mm1_body(degp_ref, x_ref, w_ref, hp_ref, dis_ref):
    deg = degp_ref[0, :, 0:1] + degp_ref[1, :, 0:1] + 1.0
    dis = lax.rsqrt(deg)
    h = jnp.dot(x_ref[...], w_ref[...], preferred_element_type=jnp.float32)
    hp_ref[...] = h * dis
    dis_ref[...] = dis


_mm1 = pl.pallas_call(
    _mm1_body,
    grid=(N // BM,),
    in_specs=[
        pl.BlockSpec((NC, BM, 16), lambda i: (0, i, 0)),
        pl.BlockSpec((BM, D), lambda i: (i, 0)),
        pl.BlockSpec((D, D), lambda i: (0, 0)),
    ],
    out_specs=[
        pl.BlockSpec((BM, D), lambda i: (i, 0)),
        pl.BlockSpec((BM, 1), lambda i: (i, 0)),
    ],
    out_shape=[
        jax.ShapeDtypeStruct((N, D), jnp.float32),
        jax.ShapeDtypeStruct((N, 1), jnp.float32),
    ],
)


def _comb_body(plo_ref, phi_ref, hp_ref, dis_ref, b_ref, out_ref):
    p = jnp.concatenate([plo_ref[0] + plo_ref[1], phi_ref[0] + phi_ref[1]],
                        axis=1)
    z = (p + hp_ref[...]) * dis_ref[...] + b_ref[...]
    out_ref[...] = jnp.maximum(z, 0.0) * dis_ref[...]


_comb = pl.pallas_call(
    _comb_body,
    grid=(N // BM,),
    in_specs=[
        pl.BlockSpec((NC, BM, DH), lambda i: (0, i, 0)),
        pl.BlockSpec((NC, BM, DH), lambda i: (0, i, 0)),
        pl.BlockSpec((BM, D), lambda i: (i, 0)),
        pl.BlockSpec((BM, 1), lambda i: (i, 0)),
        pl.BlockSpec((1, D), lambda i: (0, 0)),
    ],
    out_specs=pl.BlockSpec((BM, D), lambda i: (i, 0)),
    out_shape=jax.ShapeDtypeStruct((N, D), jnp.float32),
)


def _fin_body(plo_ref, phi_ref, hp_ref, dis_ref, w_ref, b_ref, out_ref):
    p = jnp.concatenate([plo_ref[0] + plo_ref[1], phi_ref[0] + phi_ref[1]],
                        axis=1)
    g = (p + hp_ref[...]) * dis_ref[...]
    out_ref[...] = (
        jnp.dot(g, w_ref[...], preferred_element_type=jnp.float32) + b_ref[...]
    )


_fin = pl.pallas_call(
    _fin_body,
    grid=(N // BM,),
    in_specs=[
        pl.BlockSpec((NC, BM, DH), lambda i: (0, i, 0)),
        pl.BlockSpec((NC, BM, DH), lambda i: (0, i, 0)),
        pl.BlockSpec((BM, D), lambda i: (i, 0)),
        pl.BlockSpec((BM, 1), lambda i: (i, 0)),
        pl.BlockSpec((D, D), lambda i: (0, 0)),
        pl.BlockSpec((1, D), lambda i: (0, 0)),
    ],
    out_specs=pl.BlockSpec((BM, D), lambda i: (i, 0)),
    out_shape=jax.ShapeDtypeStruct((N, D), jnp.float32),
)


def kernel(x, W1, b1, Wmu, bmu, Wls, bls, edge_index):
    src = edge_index[0].astype(jnp.int32).reshape(NW, NCHUNK, K)
    dst = edge_index[1].astype(jnp.int32).reshape(NW, NCHUNK, K)

    degp = _deg_kernel(dst)
    h1p, dis = _mm1(degp, x, W1)
    p1lo = _agg_kernel(h1p[:, :DH] + 0.0, src, dst)
    p1hi = _agg_kernel(h1p[:, DH:] + 0.0, src, dst)
    h2p = _comb(p1lo, p1hi, h1p, dis, b1.reshape(1, D))
    p2lo = _agg_kernel(h2p[:, :DH] + 0.0, src, dst)
    p2hi = _agg_kernel(h2p[:, DH:] + 0.0, src, dst)
    wcat = jnp.concatenate([Wmu, Wls], axis=1)
    bcat = jnp.concatenate([bmu, bls]).reshape(1, D)
    out = _fin(p2lo, p2hi, h2p, dis, wcat, bcat)
    return out[:, :64], out[:, 64:]


# speedup vs baseline: 19.6925x; 1.1941x over previous
import functools

import jax
import jax.numpy as jnp
from jax import lax
from jax.experimental import pallas as pl
from jax.experimental.pallas import tpu as pltpu
from jax.experimental.pallas import tpu_sc as plsc

N = 10000
E = 320000
D = 128
NC = 2
NS = 16
NW = NC * NS
EPW = E // NW
K = 80
NCHUNK = EPW // K
NP = 10240
RPT = NP // NS
ZROWS = 128
BM = 1000

_mesh = plsc.VectorSubcoreMesh(core_axis_name="c", subcore_axis_name="s")


def _ramp_into(ridx_v, off):
    @pl.loop(0, ZROWS // 16)
    def _ramp(jj):
        ridx_v[pl.ds(jj * 16, 16)] = lax.iota(jnp.int32, 16) + (off + jj * 16)


@functools.partial(
    pl.kernel,
    out_type=jax.ShapeDtypeStruct((NC, NP, 16), jnp.float32),
    mesh=_mesh,
    scratch_types=[
        pltpu.VMEM((K, 16), jnp.float32),
        pltpu.VMEM((ZROWS, 16), jnp.float32),
        pltpu.VMEM((ZROWS,), jnp.int32),
        pltpu.VMEM((NCHUNK, K), jnp.int32),
        pltpu.VMEM_SHARED((NP, 16), jnp.float32),
        pltpu.SemaphoreType.DMA,
    ],
)
def _deg_kernel(dst_hbm, out_hbm, ones_v, buf_v, ridx_v, didx_v, acc_sh, sem):
    c = lax.axis_index("c")
    s = lax.axis_index("s")
    wid = c * NS + s

    pltpu.sync_copy(dst_hbm.at[wid], didx_v)

    @pl.loop(0, K)
    def _fill_ones(j):
        ones_v[j] = jnp.full((16,), 1.0, jnp.float32)

    @pl.loop(0, ZROWS)
    def _fill_zero(j):
        buf_v[j] = jnp.zeros((16,), jnp.float32)

    @pl.loop(0, RPT // ZROWS)
    def _zero_acc(j):
        _ramp_into(ridx_v, s * RPT + j * ZROWS)
        pltpu.sync_copy(buf_v, acc_sh.at[ridx_v])

    plsc.subcore_barrier()

    @pl.loop(0, NCHUNK)
    def _chunk(ci):
        pltpu.sync_copy(ones_v, acc_sh.at[didx_v.at[ci]], add=True)

    plsc.subcore_barrier()

    @pl.loop(0, RPT // ZROWS)
    def _writeout(j):
        _ramp_into(ridx_v, s * RPT + j * ZROWS)
        pltpu.async_copy(acc_sh.at[ridx_v], buf_v, sem).wait()
        pltpu.sync_copy(buf_v, out_hbm.at[c, pl.ds(s * RPT + j * ZROWS, ZROWS)])


DH = D // 2


@functools.partial(
    pl.kernel,
    out_type=jax.ShapeDtypeStruct((NC, NP, DH), jnp.float32),
    mesh=_mesh,
    scratch_types=[
        pltpu.VMEM((NCHUNK, K), jnp.int32),
        pltpu.VMEM((NCHUNK, K), jnp.int32),
        pltpu.VMEM((4, K, DH), jnp.float32),
        pltpu.VMEM((ZROWS, DH), jnp.float32),
        pltpu.VMEM((ZROWS,), jnp.int32),
        pltpu.VMEM_SHARED((NP, DH), jnp.float32),
        pltpu.SemaphoreType.DMA,
        pltpu.SemaphoreType.DMA,
    ],
    compiler_params=pltpu.CompilerParams(use_tc_tiling_on_sc=False),
)
def _agg_kernel(h_hbm, src_hbm, dst_hbm, out_hbm,
                sidx_v, didx_v, rows_v, buf_v, ridx_v, acc_sh, sem, sem_a):
    c = lax.axis_index("c")
    s = lax.axis_index("s")
    wid = c * NS + s

    pltpu.sync_copy(src_hbm.at[wid], sidx_v)
    pltpu.sync_copy(dst_hbm.at[wid], didx_v)

    @pl.loop(0, ZROWS)
    def _fill_zero(j):
        @pl.loop(0, DH // 16)
        def _fill_row(jj):
            buf_v[j, pl.ds(jj * 16, 16)] = jnp.zeros((16,), jnp.float32)

    @pl.loop(0, RPT // ZROWS)
    def _zero_acc(j):
        _ramp_into(ridx_v, s * RPT + j * ZROWS)
        pltpu.sync_copy(buf_v, acc_sh.at[ridx_v])

    plsc.subcore_barrier()

    pltpu.async_copy(h_hbm.at[sidx_v.at[0]], rows_v.at[0], sem)

    @pl.loop(0, NCHUNK // 4)
    def _chunk(ci4):
        for b in range(4):
            ci = ci4 * 4 + b
            pltpu.make_async_copy(
                h_hbm.at[sidx_v.at[ci]], rows_v.at[b], sem).wait()
            pltpu.async_copy(
                h_hbm.at[sidx_v.at[ci + 1]], rows_v.at[(b + 1) % 4], sem)
            pltpu.async_copy(
                rows_v.at[b], acc_sh.at[didx_v.at[ci]], sem_a, add=True).wait()

    _ci = NCHUNK - 1
    pltpu.make_async_copy(h_hbm.at[sidx_v.at[_ci]], rows_v.at[0], sem).wait()
    pltpu.async_copy(
        rows_v.at[0], acc_sh.at[didx_v.at[_ci]], sem_a, add=True).wait()

    plsc.subcore_barrier()

    @pl.loop(0, RPT // ZROWS)
    def _writeout(j):
        _ramp_into(ridx_v, s * RPT + j * ZROWS)
        pltpu.async_copy(acc_sh.at[ridx_v], buf_v, sem).wait()
        pltpu.sync_copy(buf_v, out_hbm.at[c, pl.ds(s * RPT + j * ZROWS, ZROWS)])


def _mm1_body(degp_ref, x_ref, w_ref, hp_ref, dis_ref):
    deg = degp_ref[0, :, 0:1] + degp_ref[1, :, 0:1] + 1.0
    dis = lax.rsqrt(deg)
    h = jnp.dot(x_ref[...], w_ref[...], preferred_element_type=jnp.float32)
    hp_ref[...] = h * dis
    dis_ref[...] = dis


_mm1 = pl.pallas_call(
    _mm1_body,
    grid=(N // BM,),
    in_specs=[
        pl.BlockSpec((NC, BM, 16), lambda i: (0, i, 0)),
        pl.BlockSpec((BM, D), lambda i: (i, 0)),
        pl.BlockSpec((D, D), lambda i: (0, 0)),
    ],
    out_specs=[
        pl.BlockSpec((BM, D), lambda i: (i, 0)),
        pl.BlockSpec((BM, 1), lambda i: (i, 0)),
    ],
    out_shape=[
        jax.ShapeDtypeStruct((N, D), jnp.float32),
        jax.ShapeDtypeStruct((N, 1), jnp.float32),
    ],
)


def _comb_body(plo_ref, phi_ref, hp_ref, dis_ref, b_ref, out_ref):
    p = jnp.concatenate([plo_ref[0] + plo_ref[1], phi_ref[0] + phi_ref[1]],
                        axis=1)
    z = (p + hp_ref[...]) * dis_ref[...] + b_ref[...]
    out_ref[...] = jnp.maximum(z, 0.0) * dis_ref[...]


_comb = pl.pallas_call(
    _comb_body,
    grid=(N // BM,),
    in_specs=[
        pl.BlockSpec((NC, BM, DH), lambda i: (0, i, 0)),
        pl.BlockSpec((NC, BM, DH), lambda i: (0, i, 0)),
        pl.BlockSpec((BM, D), lambda i: (i, 0)),
        pl.BlockSpec((BM, 1), lambda i: (i, 0)),
        pl.BlockSpec((1, D), lambda i: (0, 0)),
    ],
    out_specs=pl.BlockSpec((BM, D), lambda i: (i, 0)),
    out_shape=jax.ShapeDtypeStruct((N, D), jnp.float32),
)


def _fin_body(plo_ref, phi_ref, hp_ref, dis_ref, w_ref, b_ref, out_ref):
    p = jnp.concatenate([plo_ref[0] + plo_ref[1], phi_ref[0] + phi_ref[1]],
                        axis=1)
    g = (p + hp_ref[...]) * dis_ref[...]
    out_ref[...] = (
        jnp.dot(g, w_ref[...], preferred_element_type=jnp.float32) + b_ref[...]
    )


_fin = pl.pallas_call(
    _fin_body,
    grid=(N // BM,),
    in_specs=[
        pl.BlockSpec((NC, BM, DH), lambda i: (0, i, 0)),
        pl.BlockSpec((NC, BM, DH), lambda i: (0, i, 0)),
        pl.BlockSpec((BM, D), lambda i: (i, 0)),
        pl.BlockSpec((BM, 1), lambda i: (i, 0)),
        pl.BlockSpec((D, D), lambda i: (0, 0)),
        pl.BlockSpec((1, D), lambda i: (0, 0)),
    ],
    out_specs=pl.BlockSpec((BM, D), lambda i: (i, 0)),
    out_shape=jax.ShapeDtypeStruct((N, D), jnp.float32),
)


def kernel(x, W1, b1, Wmu, bmu, Wls, bls, edge_index):
    src = edge_index[0].astype(jnp.int32).reshape(NW, NCHUNK, K)
    dst = edge_index[1].astype(jnp.int32).reshape(NW, NCHUNK, K)

    degp = _deg_kernel(dst)
    h1p, dis = _mm1(degp, x, W1)
    p1lo = _agg_kernel(h1p[:, :DH] + 0.0, src, dst)
    p1hi = _agg_kernel(h1p[:, DH:] + 0.0, src, dst)
    h2p = _comb(p1lo, p1hi, h1p, dis, b1.reshape(1, D))
    p2lo = _agg_kernel(h2p[:, :DH] + 0.0, src, dst)
    p2hi = _agg_kernel(h2p[:, DH:] + 0.0, src, dst)
    wcat = jnp.concatenate([Wmu, Wls], axis=1)
    bcat = jnp.concatenate([bmu, bls]).reshape(1, D)
    out = _fin(p2lo, p2hi, h2p, dis, wcat, bcat)
    return out[:, :64], out[:, 64:]


# speedup vs baseline: 20.6168x; 1.0469x over previous
import functools

import jax
import jax.numpy as jnp
from jax import lax
from jax.experimental import pallas as pl
from jax.experimental.pallas import tpu as pltpu
from jax.experimental.pallas import tpu_sc as plsc

N = 10000
E = 320000
D = 128
NC = 2
NS = 16
NW = NC * NS
EPW = E // NW
K = 80
NCHUNK = EPW // K
EPW2 = E // NS
NCHUNK2 = EPW2 // K
NP = 10240
RPT = NP // NS
ZROWS = 128
BM = 1000

_mesh = plsc.VectorSubcoreMesh(core_axis_name="c", subcore_axis_name="s")


def _ramp_into(ridx_v, off):
    @pl.loop(0, ZROWS // 16)
    def _ramp(jj):
        ridx_v[pl.ds(jj * 16, 16)] = lax.iota(jnp.int32, 16) + (off + jj * 16)


@functools.partial(
    pl.kernel,
    out_type=jax.ShapeDtypeStruct((NC, NP, 16), jnp.float32),
    mesh=_mesh,
    scratch_types=[
        pltpu.VMEM((K, 16), jnp.float32),
        pltpu.VMEM((ZROWS, 16), jnp.float32),
        pltpu.VMEM((ZROWS,), jnp.int32),
        pltpu.VMEM((NCHUNK2, K), jnp.int32),
        pltpu.VMEM_SHARED((NP, 16), jnp.float32),
        pltpu.SemaphoreType.DMA,
    ],
)
def _deg_kernel(dst_hbm, out_hbm, ones_v, buf_v, ridx_v, didx_v, acc_sh, sem):
    c = lax.axis_index("c")
    s = lax.axis_index("s")

    pltpu.sync_copy(dst_hbm.at[s], didx_v)

    @pl.loop(0, K)
    def _fill_ones(j):
        ones_v[j] = jnp.full((16,), 1.0, jnp.float32)

    @pl.loop(0, ZROWS)
    def _fill_zero(j):
        buf_v[j] = jnp.zeros((16,), jnp.float32)

    @pl.loop(0, RPT // ZROWS)
    def _zero_acc(j):
        _ramp_into(ridx_v, s * RPT + j * ZROWS)
        pltpu.sync_copy(buf_v, acc_sh.at[ridx_v])

    plsc.subcore_barrier()

    @pl.loop(0, NCHUNK)
    def _chunk(ci):
        pltpu.sync_copy(ones_v, acc_sh.at[didx_v.at[c * NCHUNK + ci]], add=True)

    plsc.subcore_barrier()

    @pl.loop(0, RPT // ZROWS)
    def _writeout(j):
        _ramp_into(ridx_v, s * RPT + j * ZROWS)
        pltpu.async_copy(acc_sh.at[ridx_v], buf_v, sem).wait()
        pltpu.sync_copy(buf_v, out_hbm.at[c, pl.ds(s * RPT + j * ZROWS, ZROWS)])


DH = D // 2


@functools.partial(
    pl.kernel,
    out_type=jax.ShapeDtypeStruct((NC, NP, DH), jnp.float32),
    mesh=_mesh,
    scratch_types=[
        pltpu.VMEM((NCHUNK2, K), jnp.int32),
        pltpu.VMEM((NCHUNK2, K), jnp.int32),
        pltpu.VMEM((4, K, DH), jnp.float32),
        pltpu.VMEM((ZROWS, DH), jnp.float32),
        pltpu.VMEM((ZROWS,), jnp.int32),
        pltpu.VMEM_SHARED((NP, DH), jnp.float32),
        pltpu.SemaphoreType.DMA,
        pltpu.SemaphoreType.DMA,
    ],
    compiler_params=pltpu.CompilerParams(use_tc_tiling_on_sc=False),
)
def _agg_kernel(hlo_hbm, hhi_hbm, src_hbm, dst_hbm, out_hbm,
                sidx_v, didx_v, rows_v, buf_v, ridx_v, acc_sh, sem, sem_a):
    c = lax.axis_index("c")
    s = lax.axis_index("s")

    pltpu.sync_copy(src_hbm.at[s], sidx_v)
    pltpu.sync_copy(dst_hbm.at[s], didx_v)

    @pl.loop(0, ZROWS)
    def _fill_zero(j):
        @pl.loop(0, DH // 16)
        def _fill_row(jj):
            buf_v[j, pl.ds(jj * 16, 16)] = jnp.zeros((16,), jnp.float32)

    @pl.loop(0, RPT // ZROWS)
    def _zero_acc(j):
        _ramp_into(ridx_v, s * RPT + j * ZROWS)
        pltpu.sync_copy(buf_v, acc_sh.at[ridx_v])

    plsc.subcore_barrier()

    def _edge_loop(h_hbm):
        pltpu.async_copy(h_hbm.at[sidx_v.at[0]], rows_v.at[0], sem)

        @pl.loop(0, NCHUNK2 // 4)
        def _chunk(ci4):
            for b in range(4):
                ci = ci4 * 4 + b
                pltpu.make_async_copy(
                    h_hbm.at[sidx_v.at[ci]], rows_v.at[b], sem).wait()
                pltpu.async_copy(
                    h_hbm.at[sidx_v.at[ci + 1]], rows_v.at[(b + 1) % 4], sem)
                pltpu.async_copy(
                    rows_v.at[b], acc_sh.at[didx_v.at[ci]], sem_a,
                    add=True).wait()

        _c0 = NCHUNK2 - 2
        _c1 = NCHUNK2 - 1
        pltpu.make_async_copy(h_hbm.at[sidx_v.at[_c0]], rows_v.at[0], sem).wait()
        pltpu.async_copy(h_hbm.at[sidx_v.at[_c1]], rows_v.at[1], sem)
        pltpu.async_copy(
            rows_v.at[0], acc_sh.at[didx_v.at[_c0]], sem_a, add=True).wait()
        pltpu.make_async_copy(h_hbm.at[sidx_v.at[_c1]], rows_v.at[1], sem).wait()
        pltpu.async_copy(
            rows_v.at[1], acc_sh.at[didx_v.at[_c1]], sem_a, add=True).wait()

    @pl.when(c == 0)
    def _lo():
        _edge_loop(hlo_hbm)

    @pl.when(c == 1)
    def _hi():
        _edge_loop(hhi_hbm)

    plsc.subcore_barrier()

    @pl.loop(0, RPT // ZROWS)
    def _writeout(j):
        _ramp_into(ridx_v, s * RPT + j * ZROWS)
        pltpu.async_copy(acc_sh.at[ridx_v], buf_v, sem).wait()
        pltpu.sync_copy(buf_v, out_hbm.at[c, pl.ds(s * RPT + j * ZROWS, ZROWS)])


def _mm1_body(degp_ref, x_ref, w_ref, hp_ref, dis_ref):
    deg = degp_ref[0, :, 0:1] + degp_ref[1, :, 0:1] + 1.0
    dis = lax.rsqrt(deg)
    h = jnp.dot(x_ref[...], w_ref[...], preferred_element_type=jnp.float32)
    hp_ref[...] = h * dis
    dis_ref[...] = dis


_mm1 = pl.pallas_call(
    _mm1_body,
    grid=(N // BM,),
    in_specs=[
        pl.BlockSpec((NC, BM, 16), lambda i: (0, i, 0)),
        pl.BlockSpec((BM, D), lambda i: (i, 0)),
        pl.BlockSpec((D, D), lambda i: (0, 0)),
    ],
    out_specs=[
        pl.BlockSpec((BM, D), lambda i: (i, 0)),
        pl.BlockSpec((BM, 1), lambda i: (i, 0)),
    ],
    out_shape=[
        jax.ShapeDtypeStruct((N, D), jnp.float32),
        jax.ShapeDtypeStruct((N, 1), jnp.float32),
    ],
)


def _comb_body(p_ref, hp_ref, dis_ref, b_ref, out_ref):
    p = jnp.concatenate([p_ref[0], p_ref[1]], axis=1)
    z = (p + hp_ref[...]) * dis_ref[...] + b_ref[...]
    out_ref[...] = jnp.maximum(z, 0.0) * dis_ref[...]


_comb = pl.pallas_call(
    _comb_body,
    grid=(N // BM,),
    in_specs=[
        pl.BlockSpec((NC, BM, DH), lambda i: (0, i, 0)),
        pl.BlockSpec((BM, D), lambda i: (i, 0)),
        pl.BlockSpec((BM, 1), lambda i: (i, 0)),
        pl.BlockSpec((1, D), lambda i: (0, 0)),
    ],
    out_specs=pl.BlockSpec((BM, D), lambda i: (i, 0)),
    out_shape=jax.ShapeDtypeStruct((N, D), jnp.float32),
)


def _fin_body(p_ref, hp_ref, dis_ref, w_ref, b_ref, out_ref):
    p = jnp.concatenate([p_ref[0], p_ref[1]], axis=1)
    g = (p + hp_ref[...]) * dis_ref[...]
    out_ref[...] = (
        jnp.dot(g, w_ref[...], preferred_element_type=jnp.float32) + b_ref[...]
    )


_fin = pl.pallas_call(
    _fin_body,
    grid=(N // BM,),
    in_specs=[
        pl.BlockSpec((NC, BM, DH), lambda i: (0, i, 0)),
        pl.BlockSpec((BM, D), lambda i: (i, 0)),
        pl.BlockSpec((BM, 1), lambda i: (i, 0)),
        pl.BlockSpec((D, D), lambda i: (0, 0)),
        pl.BlockSpec((1, D), lambda i: (0, 0)),
    ],
    out_specs=pl.BlockSpec((BM, D), lambda i: (i, 0)),
    out_shape=jax.ShapeDtypeStruct((N, D), jnp.float32),
)


def kernel(x, W1, b1, Wmu, bmu, Wls, bls, edge_index):
    src2 = edge_index[0].astype(jnp.int32).reshape(NS, NCHUNK2, K)
    dst2 = edge_index[1].astype(jnp.int32).reshape(NS, NCHUNK2, K)

    degp = _deg_kernel(dst2)
    h1p, dis = _mm1(degp, x, W1)
    p1 = _agg_kernel(h1p[:, :DH] + 0.0, h1p[:, DH:] + 0.0, src2, dst2)
    h2p = _comb(p1, h1p, dis, b1.reshape(1, D))
    p2 = _agg_kernel(h2p[:, :DH] + 0.0, h2p[:, DH:] + 0.0, src2, dst2)
    wcat = jnp.concatenate([Wmu, Wls], axis=1)
    bcat = jnp.concatenate([bmu, bls]).reshape(1, D)
    out = _fin(p2, h2p, dis, wcat, bcat)
    return out[:, :64], out[:, 64:]


# speedup vs baseline: 30.5457x; 1.4816x over previous
import functools

import jax
import jax.numpy as jnp
from jax import lax
from jax.experimental import pallas as pl
from jax.experimental.pallas import tpu as pltpu
from jax.experimental.pallas import tpu_sc as plsc

N = 10000
E = 320000
D = 128
NC = 2
NS = 16
NW = NC * NS
EPW = E // NW
K = 80
NCHUNK = EPW // K
EPW2 = E // NS
NCHUNK2 = EPW2 // K
NP = 10240
RPT = NP // NS
ZROWS = 128
BM = 1000

_mesh = plsc.VectorSubcoreMesh(core_axis_name="c", subcore_axis_name="s")


def _ramp_into(ridx_v, off):
    @pl.loop(0, ZROWS // 16)
    def _ramp(jj):
        ridx_v[pl.ds(jj * 16, 16)] = lax.iota(jnp.int32, 16) + (off + jj * 16)


@functools.partial(
    pl.kernel,
    out_type=jax.ShapeDtypeStruct((NC, NP, 16), jnp.float32),
    mesh=_mesh,
    scratch_types=[
        pltpu.VMEM((K, 16), jnp.float32),
        pltpu.VMEM((ZROWS, 16), jnp.float32),
        pltpu.VMEM((ZROWS,), jnp.int32),
        pltpu.VMEM((NCHUNK2, K), jnp.int32),
        pltpu.VMEM_SHARED((NP, 16), jnp.float32),
        pltpu.SemaphoreType.DMA,
    ],
)
def _deg_kernel(dst_hbm, out_hbm, ones_v, buf_v, ridx_v, didx_v, acc_sh, sem):
    c = lax.axis_index("c")
    s = lax.axis_index("s")

    pltpu.sync_copy(dst_hbm.at[s], didx_v)

    @pl.loop(0, K)
    def _fill_ones(j):
        ones_v[j] = jnp.full((16,), 1.0, jnp.float32)

    @pl.loop(0, ZROWS)
    def _fill_zero(j):
        buf_v[j] = jnp.zeros((16,), jnp.float32)

    @pl.loop(0, RPT // ZROWS)
    def _zero_acc(j):
        _ramp_into(ridx_v, s * RPT + j * ZROWS)
        pltpu.sync_copy(buf_v, acc_sh.at[ridx_v])

    plsc.subcore_barrier()

    @pl.loop(0, NCHUNK)
    def _chunk(ci):
        pltpu.sync_copy(ones_v, acc_sh.at[didx_v.at[c * NCHUNK + ci]], add=True)

    plsc.subcore_barrier()

    @pl.loop(0, RPT // ZROWS)
    def _writeout(j):
        _ramp_into(ridx_v, s * RPT + j * ZROWS)
        pltpu.async_copy(acc_sh.at[ridx_v], buf_v, sem).wait()
        pltpu.sync_copy(buf_v, out_hbm.at[c, pl.ds(s * RPT + j * ZROWS, ZROWS)])


DH = D // 2


@functools.partial(
    pl.kernel,
    out_type=jax.ShapeDtypeStruct((NC, NP, DH), jnp.float32),
    mesh=_mesh,
    scratch_types=[
        pltpu.VMEM((NCHUNK2, K), jnp.int32),
        pltpu.VMEM((NCHUNK2, K), jnp.int32),
        pltpu.VMEM((4, K, DH), jnp.float32),
        pltpu.VMEM((ZROWS, DH), jnp.float32),
        pltpu.VMEM((ZROWS,), jnp.int32),
        pltpu.VMEM_SHARED((NP, DH), jnp.float32),
        pltpu.SemaphoreType.DMA,
        pltpu.SemaphoreType.DMA,
    ],
    compiler_params=pltpu.CompilerParams(use_tc_tiling_on_sc=False),
)
def _agg_kernel(hlo_hbm, hhi_hbm, src_hbm, dst_hbm, out_hbm,
                sidx_v, didx_v, rows_v, buf_v, ridx_v, acc_sh, sem, sem_a):
    c = lax.axis_index("c")
    s = lax.axis_index("s")

    pltpu.sync_copy(src_hbm.at[s], sidx_v)
    pltpu.sync_copy(dst_hbm.at[s], didx_v)

    @pl.loop(0, ZROWS)
    def _fill_zero(j):
        @pl.loop(0, DH // 16)
        def _fill_row(jj):
            buf_v[j, pl.ds(jj * 16, 16)] = jnp.zeros((16,), jnp.float32)

    @pl.loop(0, RPT // ZROWS)
    def _zero_acc(j):
        _ramp_into(ridx_v, s * RPT + j * ZROWS)
        pltpu.sync_copy(buf_v, acc_sh.at[ridx_v])

    plsc.subcore_barrier()

    def _edge_loop(h_hbm):
        pltpu.async_copy(h_hbm.at[sidx_v.at[0]], rows_v.at[0], sem)
        pltpu.async_copy(h_hbm.at[sidx_v.at[1]], rows_v.at[1], sem)

        @pl.loop(0, NCHUNK2 // 4 - 1)
        def _chunk(ci4):
            for b in range(4):
                ci = ci4 * 4 + b
                pltpu.make_async_copy(
                    h_hbm.at[sidx_v.at[ci]], rows_v.at[b], sem).wait()
                pltpu.async_copy(
                    h_hbm.at[sidx_v.at[ci + 2]], rows_v.at[(b + 2) % 4], sem)
                pltpu.async_copy(
                    rows_v.at[b], acc_sh.at[didx_v.at[ci]], sem_a,
                    add=True).wait()

        for off in range(6):
            ci = NCHUNK2 - 6 + off
            b = ci % 4
            pltpu.make_async_copy(
                h_hbm.at[sidx_v.at[ci]], rows_v.at[b], sem).wait()
            if off < 4:
                pltpu.async_copy(
                    h_hbm.at[sidx_v.at[ci + 2]], rows_v.at[(b + 2) % 4], sem)
            pltpu.async_copy(
                rows_v.at[b], acc_sh.at[didx_v.at[ci]], sem_a,
                add=True).wait()

    @pl.when(c == 0)
    def _lo():
        _edge_loop(hlo_hbm)

    @pl.when(c == 1)
    def _hi():
        _edge_loop(hhi_hbm)

    plsc.subcore_barrier()

    @pl.loop(0, RPT // ZROWS)
    def _writeout(j):
        _ramp_into(ridx_v, s * RPT + j * ZROWS)
        pltpu.async_copy(acc_sh.at[ridx_v], buf_v, sem).wait()
        pltpu.sync_copy(buf_v, out_hbm.at[c, pl.ds(s * RPT + j * ZROWS, ZROWS)])


def _mm1_body(degp_ref, x_ref, w_ref, hp_ref, dis_ref):
    deg = degp_ref[0, :, 0:1] + degp_ref[1, :, 0:1] + 1.0
    dis = lax.rsqrt(deg)
    h = jnp.dot(x_ref[...], w_ref[...], preferred_element_type=jnp.float32)
    hp_ref[...] = h * dis
    dis_ref[...] = dis


_mm1 = pl.pallas_call(
    _mm1_body,
    grid=(N // BM,),
    in_specs=[
        pl.BlockSpec((NC, BM, 16), lambda i: (0, i, 0)),
        pl.BlockSpec((BM, D), lambda i: (i, 0)),
        pl.BlockSpec((D, D), lambda i: (0, 0)),
    ],
    out_specs=[
        pl.BlockSpec((BM, D), lambda i: (i, 0)),
        pl.BlockSpec((BM, 1), lambda i: (i, 0)),
    ],
    out_shape=[
        jax.ShapeDtypeStruct((N, D), jnp.float32),
        jax.ShapeDtypeStruct((N, 1), jnp.float32),
    ],
)


def _comb_body(p_ref, hp_ref, dis_ref, b_ref, out_ref):
    p = jnp.concatenate([p_ref[0], p_ref[1]], axis=1)
    z = (p + hp_ref[...]) * dis_ref[...] + b_ref[...]
    out_ref[...] = jnp.maximum(z, 0.0) * dis_ref[...]


_comb = pl.pallas_call(
    _comb_body,
    grid=(N // BM,),
    in_specs=[
        pl.BlockSpec((NC, BM, DH), lambda i: (0, i, 0)),
        pl.BlockSpec((BM, D), lambda i: (i, 0)),
        pl.BlockSpec((BM, 1), lambda i: (i, 0)),
        pl.BlockSpec((1, D), lambda i: (0, 0)),
    ],
    out_specs=pl.BlockSpec((BM, D), lambda i: (i, 0)),
    out_shape=jax.ShapeDtypeStruct((N, D), jnp.float32),
)


def _fin_body(p_ref, hp_ref, dis_ref, w_ref, b_ref, out_ref):
    p = jnp.concatenate([p_ref[0], p_ref[1]], axis=1)
    g = (p + hp_ref[...]) * dis_ref[...]
    out_ref[...] = (
        jnp.dot(g, w_ref[...], preferred_element_type=jnp.float32) + b_ref[...]
    )


_fin = pl.pallas_call(
    _fin_body,
    grid=(N // BM,),
    in_specs=[
        pl.BlockSpec((NC, BM, DH), lambda i: (0, i, 0)),
        pl.BlockSpec((BM, D), lambda i: (i, 0)),
        pl.BlockSpec((BM, 1), lambda i: (i, 0)),
        pl.BlockSpec((D, D), lambda i: (0, 0)),
        pl.BlockSpec((1, D), lambda i: (0, 0)),
    ],
    out_specs=pl.BlockSpec((BM, D), lambda i: (i, 0)),
    out_shape=jax.ShapeDtypeStruct((N, D), jnp.float32),
)


def kernel(x, W1, b1, Wmu, bmu, Wls, bls, edge_index):
    src2 = edge_index[0].astype(jnp.int32).reshape(NS, NCHUNK2, K)
    dst2 = edge_index[1].astype(jnp.int32).reshape(NS, NCHUNK2, K)

    degp = _deg_kernel(dst2)
    h1p, dis = _mm1(degp, x, W1)
    p1 = _agg_kernel(h1p[:, :DH] + 0.0, h1p[:, DH:] + 0.0, src2, dst2)
    h2p = _comb(p1, h1p, dis, b1.reshape(1, D))
    p2 = _agg_kernel(h2p[:, :DH] + 0.0, h2p[:, DH:] + 0.0, src2, dst2)
    wcat = jnp.concatenate([Wmu, Wls], axis=1)
    bcat = jnp.concatenate([bmu, bls]).reshape(1, D)
    out = _fin(p2, h2p, dis, wcat, bcat)
    return out[:, :64], out[:, 64:]


# speedup vs baseline: 35.5147x; 1.1627x over previous
import functools

import jax
import jax.numpy as jnp
from jax import lax
from jax.experimental import pallas as pl
from jax.experimental.pallas import tpu as pltpu
from jax.experimental.pallas import tpu_sc as plsc

N = 10000
E = 320000
D = 128
NC = 2
NS = 16
NW = NC * NS
EPW = E // NW
K = 80
NCHUNK = EPW // K
EPW2 = E // NS
NCHUNK2 = EPW2 // K
NP = 10240
RPT = NP // NS
ZROWS = 128
BM = 1000

_mesh = plsc.VectorSubcoreMesh(core_axis_name="c", subcore_axis_name="s")


def _ramp_into(ridx_v, off):
    @pl.loop(0, ZROWS // 16)
    def _ramp(jj):
        ridx_v[pl.ds(jj * 16, 16)] = lax.iota(jnp.int32, 16) + (off + jj * 16)


@functools.partial(
    pl.kernel,
    out_type=jax.ShapeDtypeStruct((NC, NP, 16), jnp.float32),
    mesh=_mesh,
    scratch_types=[
        pltpu.VMEM((K, 16), jnp.float32),
        pltpu.VMEM((ZROWS, 16), jnp.float32),
        pltpu.VMEM((ZROWS,), jnp.int32),
        pltpu.VMEM((NCHUNK2, K), jnp.int32),
        pltpu.VMEM_SHARED((NP, 16), jnp.float32),
        pltpu.SemaphoreType.DMA,
    ],
)
def _deg_kernel(dst_hbm, out_hbm, ones_v, buf_v, ridx_v, didx_v, acc_sh, sem):
    c = lax.axis_index("c")
    s = lax.axis_index("s")

    pltpu.sync_copy(dst_hbm.at[s], didx_v)

    @pl.loop(0, K)
    def _fill_ones(j):
        ones_v[j] = jnp.full((16,), 1.0, jnp.float32)

    @pl.loop(0, ZROWS)
    def _fill_zero(j):
        buf_v[j] = jnp.zeros((16,), jnp.float32)

    @pl.loop(0, RPT // ZROWS)
    def _zero_acc(j):
        _ramp_into(ridx_v, s * RPT + j * ZROWS)
        pltpu.sync_copy(buf_v, acc_sh.at[ridx_v])

    plsc.subcore_barrier()

    @pl.loop(0, NCHUNK)
    def _chunk(ci):
        pltpu.sync_copy(ones_v, acc_sh.at[didx_v.at[c * NCHUNK + ci]], add=True)

    plsc.subcore_barrier()

    @pl.loop(0, RPT // ZROWS)
    def _writeout(j):
        _ramp_into(ridx_v, s * RPT + j * ZROWS)
        pltpu.async_copy(acc_sh.at[ridx_v], buf_v, sem).wait()
        pltpu.sync_copy(buf_v, out_hbm.at[c, pl.ds(s * RPT + j * ZROWS, ZROWS)])


DH = D // 2


@functools.partial(
    pl.kernel,
    out_type=jax.ShapeDtypeStruct((NC, NP, DH), jnp.float32),
    mesh=_mesh,
    scratch_types=[
        pltpu.VMEM((NCHUNK2, K), jnp.int32),
        pltpu.VMEM((NCHUNK2, K), jnp.int32),
        pltpu.VMEM((4, K, DH), jnp.float32),
        pltpu.VMEM((ZROWS, DH), jnp.float32),
        pltpu.VMEM((ZROWS,), jnp.int32),
        pltpu.VMEM_SHARED((NP, DH), jnp.float32),
        pltpu.SemaphoreType.DMA,
        pltpu.SemaphoreType.DMA,
    ],
    compiler_params=pltpu.CompilerParams(use_tc_tiling_on_sc=False),
)
def _agg_kernel(hlo_hbm, hhi_hbm, src_hbm, dst_hbm, out_hbm,
                sidx_v, didx_v, rows_v, buf_v, ridx_v, acc_sh, sem, sem_a):
    c = lax.axis_index("c")
    s = lax.axis_index("s")

    pltpu.sync_copy(src_hbm.at[s], sidx_v)
    pltpu.sync_copy(dst_hbm.at[s], didx_v)

    @pl.loop(0, ZROWS)
    def _fill_zero(j):
        @pl.loop(0, DH // 16)
        def _fill_row(jj):
            buf_v[j, pl.ds(jj * 16, 16)] = jnp.zeros((16,), jnp.float32)

    @pl.loop(0, RPT // ZROWS)
    def _zero_acc(j):
        _ramp_into(ridx_v, s * RPT + j * ZROWS)
        pltpu.sync_copy(buf_v, acc_sh.at[ridx_v])

    plsc.subcore_barrier()

    def _edge_loop(h_hbm):
        pltpu.async_copy(h_hbm.at[sidx_v.at[0]], rows_v.at[0], sem)
        pltpu.async_copy(h_hbm.at[sidx_v.at[1]], rows_v.at[1], sem)
        pltpu.async_copy(h_hbm.at[sidx_v.at[2]], rows_v.at[2], sem)

        @pl.loop(0, NCHUNK2 // 4 - 1)
        def _chunk(ci4):
            for b in range(4):
                ci = ci4 * 4 + b
                pltpu.make_async_copy(
                    h_hbm.at[sidx_v.at[ci]], rows_v.at[b], sem).wait()
                pltpu.async_copy(
                    h_hbm.at[sidx_v.at[ci + 3]], rows_v.at[(b + 3) % 4], sem)
                pltpu.async_copy(
                    rows_v.at[b], acc_sh.at[didx_v.at[ci]], sem_a,
                    add=True).wait()

        for off in range(6):
            ci = NCHUNK2 - 6 + off
            b = ci % 4
            pltpu.make_async_copy(
                h_hbm.at[sidx_v.at[ci]], rows_v.at[b], sem).wait()
            if off < 3:
                pltpu.async_copy(
                    h_hbm.at[sidx_v.at[ci + 3]], rows_v.at[(b + 3) % 4], sem)
            pltpu.async_copy(
                rows_v.at[b], acc_sh.at[didx_v.at[ci]], sem_a,
                add=True).wait()

    @pl.when(c == 0)
    def _lo():
        _edge_loop(hlo_hbm)

    @pl.when(c == 1)
    def _hi():
        _edge_loop(hhi_hbm)

    plsc.subcore_barrier()

    @pl.loop(0, RPT // ZROWS)
    def _writeout(j):
        _ramp_into(ridx_v, s * RPT + j * ZROWS)
        pltpu.async_copy(acc_sh.at[ridx_v], buf_v, sem).wait()
        pltpu.sync_copy(buf_v, out_hbm.at[c, pl.ds(s * RPT + j * ZROWS, ZROWS)])


def _mm1_body(degp_ref, x_ref, w_ref, hp_ref, dis_ref):
    deg = degp_ref[0, :, 0:1] + degp_ref[1, :, 0:1] + 1.0
    dis = lax.rsqrt(deg)
    h = jnp.dot(x_ref[...], w_ref[...], preferred_element_type=jnp.float32)
    hp_ref[...] = h * dis
    dis_ref[...] = dis


_mm1 = pl.pallas_call(
    _mm1_body,
    grid=(N // BM,),
    in_specs=[
        pl.BlockSpec((NC, BM, 16), lambda i: (0, i, 0)),
        pl.BlockSpec((BM, D), lambda i: (i, 0)),
        pl.BlockSpec((D, D), lambda i: (0, 0)),
    ],
    out_specs=[
        pl.BlockSpec((BM, D), lambda i: (i, 0)),
        pl.BlockSpec((BM, 1), lambda i: (i, 0)),
    ],
    out_shape=[
        jax.ShapeDtypeStruct((N, D), jnp.float32),
        jax.ShapeDtypeStruct((N, 1), jnp.float32),
    ],
)


def _comb_body(p_ref, hp_ref, dis_ref, b_ref, out_ref):
    p = jnp.concatenate([p_ref[0], p_ref[1]], axis=1)
    z = (p + hp_ref[...]) * dis_ref[...] + b_ref[...]
    out_ref[...] = jnp.maximum(z, 0.0) * dis_ref[...]


_comb = pl.pallas_call(
    _comb_body,
    grid=(N // BM,),
    in_specs=[
        pl.BlockSpec((NC, BM, DH), lambda i: (0, i, 0)),
        pl.BlockSpec((BM, D), lambda i: (i, 0)),
        pl.BlockSpec((BM, 1), lambda i: (i, 0)),
        pl.BlockSpec((1, D), lambda i: (0, 0)),
    ],
    out_specs=pl.BlockSpec((BM, D), lambda i: (i, 0)),
    out_shape=jax.ShapeDtypeStruct((N, D), jnp.float32),
)


def _fin_body(p_ref, hp_ref, dis_ref, w_ref, b_ref, out_ref):
    p = jnp.concatenate([p_ref[0], p_ref[1]], axis=1)
    g = (p + hp_ref[...]) * dis_ref[...]
    out_ref[...] = (
        jnp.dot(g, w_ref[...], preferred_element_type=jnp.float32) + b_ref[...]
    )


_fin = pl.pallas_call(
    _fin_body,
    grid=(N // BM,),
    in_specs=[
        pl.BlockSpec((NC, BM, DH), lambda i: (0, i, 0)),
        pl.BlockSpec((BM, D), lambda i: (i, 0)),
        pl.BlockSpec((BM, 1), lambda i: (i, 0)),
        pl.BlockSpec((D, D), lambda i: (0, 0)),
        pl.BlockSpec((1, D), lambda i: (0, 0)),
    ],
    out_specs=pl.BlockSpec((BM, D), lambda i: (i, 0)),
    out_shape=jax.ShapeDtypeStruct((N, D), jnp.float32),
)


def kernel(x, W1, b1, Wmu, bmu, Wls, bls, edge_index):
    src2 = edge_index[0].astype(jnp.int32).reshape(NS, NCHUNK2, K)
    dst2 = edge_index[1].astype(jnp.int32).reshape(NS, NCHUNK2, K)

    degp = _deg_kernel(dst2)
    h1p, dis = _mm1(degp, x, W1)
    p1 = _agg_kernel(h1p[:, :DH] + 0.0, h1p[:, DH:] + 0.0, src2, dst2)
    h2p = _comb(p1, h1p, dis, b1.reshape(1, D))
    p2 = _agg_kernel(h2p[:, :DH] + 0.0, h2p[:, DH:] + 0.0, src2, dst2)
    wcat = jnp.concatenate([Wmu, Wls], axis=1)
    bcat = jnp.concatenate([bmu, bls]).reshape(1, D)
    out = _fin(p2, h2p, dis, wcat, bcat)
    return out[:, :64], out[:, 64:]
